# baseline (speedup 1.0000x reference)
import jax
import jax.numpy as jnp
from jax.experimental import pallas as pl
from jax.experimental.pallas import tpu as pltpu
from jax.experimental.pallas import tpu_sc as plsc

_B, _L, _J, _D = 4, 2048, 8, 256
_DX, _TED = 6, 6
_NS = _DX * (_TED - 1)

_NW = 32
_ROWS = _B * _J * _L
_RPW = _ROWS // _NW
_K = 128
_NDMA = _RPW // _K


_DN0 = (((0,), (0,)), ((), ()))


def _vt_kernel(x_ref, y_ref, a_ref, ss_ref, bs_ref, w1s_ref, w2_ref,
               vtb_ref, local_ref, out_vt_ref, base_ref):
    j = pl.program_id(1)

    @pl.when(j == 0)
    def _():
        xt = jnp.nan_to_num(x_ref[0])
        lin = jax.lax.dot_general(xt, a_ref[...], _DN0,
                                  preferred_element_type=jnp.float32)
        xs_t = jax.lax.dot(ss_ref[...], xt,
                           preferred_element_type=jnp.float32) + bs_ref[...]
        te_t = jnp.sin(xs_t)
        sin_part = jax.lax.dot_general(te_t, w1s_ref[...], _DN0,
                                       preferred_element_type=jnp.float32)
        base_ref[...] = local_ref[...] + lin + sin_part + vtb_ref[...]

    ycol = y_ref[0]
    nanf = jnp.isnan(ycol).astype(jnp.float32)
    yc = jnp.where(nanf > 0, jnp.float32(0), ycol)
    a2 = jnp.concatenate([yc, nanf], axis=1)
    p = jax.lax.dot(a2, w2_ref[...],
                    preferred_element_type=jnp.float32)
    out_vt_ref[0] = base_ref[...] + p


def _space_kernel(space_hbm, out_hbm, row_v, buf_v, sem):
    c = jax.lax.axis_index("c")
    s = jax.lax.axis_index("s")
    wid = s * 2 + c
    b = wid // _J
    j = wid % _J
    pltpu.sync_copy(space_hbm.at[pl.ds(j, 1)], row_v)
    vs = [row_v[0, pl.ds(i * 16, 16)] for i in range(_D // 16)]

    def rep(k, carry):
        for i in range(_D // 16):
            buf_v[k, pl.ds(i * 16, 16)] = vs[i]
        return carry

    jax.lax.fori_loop(0, _K, rep, 0)
    copies = [
        pltpu.async_copy(
            buf_v, out_hbm.at[b, pl.ds(j * _RPW + i * _K, _K)], sem)
        for i in range(_NDMA)
    ]
    for cp in copies:
        cp.wait()


def kernel(y, x, t2v_w, t2v_b, local_table, vt_W, vt_b, given_table, space_table):
    w1 = vt_W[1:]
    a_mat = t2v_w[:, 0:1] * w1[0::_TED]
    c_lin = t2v_b[:, 0] @ w1[0::_TED]
    ss_t = (jnp.eye(_DX, dtype=jnp.float32)[:, :, None]
            * t2v_w[:, None, 1:]).reshape(_DX, _NS).T
    bs_t = t2v_b[:, 1:].reshape(_NS, 1)
    sin_rows = (jnp.arange(_DX * _TED).reshape(_DX, _TED)[:, 1:]).reshape(-1)
    w1s = w1[sin_rows]
    vtb2 = (vt_b + c_lin + given_table[1]).reshape(1, _D)
    w2 = jnp.concatenate(
        [vt_W[0:1], (given_table[0] - given_table[1]).reshape(1, _D)], axis=0)
    y_t = jnp.transpose(y, (0, 2, 1)).reshape(_B * _J, _L, 1)
    x_t = jnp.transpose(x, (0, 2, 1))

    grid = (_B, _J)
    out_vt = pl.pallas_call(
        _vt_kernel,
        grid=grid,
        in_specs=[
            pl.BlockSpec((1, _DX, _L), lambda b, j: (b, 0, 0)),
            pl.BlockSpec((1, _L, 1), lambda b, j: (b * _J + j, 0, 0)),
            pl.BlockSpec((_DX, _D), lambda b, j: (0, 0)),
            pl.BlockSpec((_NS, _DX), lambda b, j: (0, 0)),
            pl.BlockSpec((_NS, 1), lambda b, j: (0, 0)),
            pl.BlockSpec((_NS, _D), lambda b, j: (0, 0)),
            pl.BlockSpec((2, _D), lambda b, j: (0, 0)),
            pl.BlockSpec((1, _D), lambda b, j: (0, 0)),
            pl.BlockSpec((_L, _D), lambda b, j: (0, 0)),
        ],
        out_specs=pl.BlockSpec((1, _L, _D), lambda b, j: (b, j, 0)),
        out_shape=jax.ShapeDtypeStruct((_B, _J * _L, _D), jnp.float32),
        scratch_shapes=[pltpu.VMEM((_L, _D), jnp.float32)],
        compiler_params=pltpu.CompilerParams(
            dimension_semantics=("parallel", "arbitrary")),
    )(x_t, y_t, a_mat, ss_t, bs_t, w1s, w2, vtb2, local_table[:_L])

    sc_call = pl.kernel(
        _space_kernel,
        out_type=jax.ShapeDtypeStruct((_B, _J * _L, _D), jnp.float32),
        mesh=plsc.VectorSubcoreMesh(core_axis_name="c", subcore_axis_name="s"),
        scratch_types=[
            pltpu.VMEM((1, _D), jnp.float32),
            pltpu.VMEM((_K, _D), jnp.float32),
            pltpu.SemaphoreType.DMA,
        ],
    )
    out_sp = sc_call(space_table)

    var_idx = jnp.broadcast_to(
        jnp.repeat(jnp.arange(_J, dtype=jnp.int32), _L)[None, :],
        (_B, _J * _L))
    return (out_vt, out_sp, var_idx)


# speedup vs baseline: 1.0100x; 1.0100x over previous
import jax
import jax.numpy as jnp
from jax.experimental import pallas as pl
from jax.experimental.pallas import tpu as pltpu
from jax.experimental.pallas import tpu_sc as plsc

_B, _L, _J, _D = 4, 2048, 8, 256
_DX, _TED = 6, 6
_NS = _DX * (_TED - 1)

_NW = 32
_ROWS = _B * _J * _L
_RPW = _ROWS // _NW
_K = 128
_NDMA = _RPW // _K


_DN0 = (((0,), (0,)), ((), ()))


def _vt_kernel(x_ref, y_ref, a_ref, ss_ref, bs_ref, w1s_ref, w2_ref,
               vtb_ref, local_ref, out_vt_ref, base_ref):
    j = pl.program_id(1)

    @pl.when(j == 0)
    def _():
        xt = jnp.nan_to_num(x_ref[0])
        lin = jax.lax.dot_general(xt, a_ref[...], _DN0,
                                  preferred_element_type=jnp.float32)
        xs_t = jax.lax.dot(ss_ref[...], xt,
                           preferred_element_type=jnp.float32) + bs_ref[...]
        te_t = jnp.sin(xs_t)
        sin_part = jax.lax.dot_general(te_t, w1s_ref[...], _DN0,
                                       preferred_element_type=jnp.float32)
        base_ref[...] = local_ref[...] + lin + sin_part + vtb_ref[...]

    ycol = y_ref[0]
    nanf = jnp.isnan(ycol).astype(jnp.float32)
    yc = jnp.where(nanf > 0, jnp.float32(0), ycol)
    a2 = jnp.concatenate([yc, nanf], axis=1)
    p = jax.lax.dot(a2, w2_ref[...],
                    preferred_element_type=jnp.float32)
    out_vt_ref[0] = base_ref[...] + p


def _space_kernel(space_hbm, out_hbm, row_v, buf_v, sem):
    c = jax.lax.axis_index("c")
    s = jax.lax.axis_index("s")
    wid = s * 2 + c
    b = wid // _J
    j = wid % _J
    pltpu.sync_copy(space_hbm.at[pl.ds(j, 1)], row_v)
    vs = [row_v[0, pl.ds(i * 16, 16)] for i in range(_D // 16)]

    def rep(k, carry):
        for i in range(_D // 16):
            buf_v[k, pl.ds(i * 16, 16)] = vs[i]
        return carry

    jax.lax.fori_loop(0, _K, rep, 0)
    copies = [
        pltpu.async_copy(
            buf_v, out_hbm.at[b, pl.ds(j * _RPW + i * _K, _K)], sem)
        for i in range(_NDMA)
    ]
    for cp in copies:
        cp.wait()


def kernel(y, x, t2v_w, t2v_b, local_table, vt_W, vt_b, given_table, space_table):
    w1 = vt_W[1:]
    a_mat = t2v_w[:, 0:1] * w1[0::_TED]
    c_lin = t2v_b[:, 0] @ w1[0::_TED]
    ss_t = (jnp.eye(_DX, dtype=jnp.float32)[:, :, None]
            * t2v_w[:, None, 1:]).reshape(_DX, _NS).T
    bs_t = t2v_b[:, 1:].reshape(_NS, 1)
    sin_rows = (jnp.arange(_DX * _TED).reshape(_DX, _TED)[:, 1:]).reshape(-1)
    w1s = w1[sin_rows]
    vtb2 = (vt_b + c_lin + given_table[1]).reshape(1, _D)
    w2 = jnp.concatenate(
        [vt_W[0:1], (given_table[0] - given_table[1]).reshape(1, _D)], axis=0)
    y_t = jnp.transpose(y, (0, 2, 1)).reshape(_B * _J, _L, 1)
    x_t = jnp.transpose(x, (0, 2, 1))

    grid = (_B, _J)
    out_vt = pl.pallas_call(
        _vt_kernel,
        grid=grid,
        in_specs=[
            pl.BlockSpec((1, _DX, _L), lambda b, j: (b, 0, 0)),
            pl.BlockSpec((1, _L, 1), lambda b, j: (b * _J + j, 0, 0)),
            pl.BlockSpec((_DX, _D), lambda b, j: (0, 0)),
            pl.BlockSpec((_NS, _DX), lambda b, j: (0, 0)),
            pl.BlockSpec((_NS, 1), lambda b, j: (0, 0)),
            pl.BlockSpec((_NS, _D), lambda b, j: (0, 0)),
            pl.BlockSpec((2, _D), lambda b, j: (0, 0)),
            pl.BlockSpec((1, _D), lambda b, j: (0, 0)),
            pl.BlockSpec((_L, _D), lambda b, j: (0, 0)),
        ],
        out_specs=pl.BlockSpec((1, _L, _D), lambda b, j: (b, j, 0)),
        out_shape=jax.ShapeDtypeStruct((_B, _J * _L, _D), jnp.float32),
        scratch_shapes=[pltpu.VMEM((_L, _D), jnp.float32)],
        compiler_params=pltpu.CompilerParams(
            dimension_semantics=("parallel", "arbitrary")),
    )(x_t, y_t, a_mat, ss_t, bs_t, w1s, w2, vtb2, local_table)

    sc_call = pl.kernel(
        _space_kernel,
        out_type=jax.ShapeDtypeStruct((_B, _J * _L, _D), jnp.float32),
        mesh=plsc.VectorSubcoreMesh(core_axis_name="c", subcore_axis_name="s"),
        scratch_types=[
            pltpu.VMEM((1, _D), jnp.float32),
            pltpu.VMEM((_K, _D), jnp.float32),
            pltpu.SemaphoreType.DMA,
        ],
    )
    out_sp = sc_call(space_table)

    var_idx = jnp.broadcast_to(
        jnp.repeat(jnp.arange(_J, dtype=jnp.int32), _L)[None, :],
        (_B, _J * _L))
    return (out_vt, out_sp, var_idx)


# speedup vs baseline: 1.3435x; 1.3301x over previous
import jax
import jax.numpy as jnp
from jax.experimental import pallas as pl
from jax.experimental.pallas import tpu as pltpu
from jax.experimental.pallas import tpu_sc as plsc

_B, _L, _J, _D = 4, 2048, 8, 256
_DX, _TED = 6, 6
_NS = _DX * (_TED - 1)

_NW = 32
_ROWS = _B * _J * _L
_RPW = _ROWS // _NW
_K = 128
_NDMA = _RPW // _K


_DN0 = (((0,), (0,)), ((), ()))


def _vt_kernel(x_ref, y_ref, a_ref, ss_ref, bs_ref, w1s_ref, w2_ref,
               vtb_ref, local_ref, out_vt_ref, base_ref):
    j = pl.program_id(1)

    @pl.when(j == 0)
    def _():
        xt = jnp.nan_to_num(x_ref[0])
        lin = jax.lax.dot_general(xt, a_ref[...], _DN0,
                                  preferred_element_type=jnp.float32)
        xs_t = jax.lax.dot(ss_ref[...], xt,
                           preferred_element_type=jnp.float32) + bs_ref[...]
        te_t = jnp.sin(xs_t)
        sin_part = jax.lax.dot_general(te_t, w1s_ref[...], _DN0,
                                       preferred_element_type=jnp.float32)
        base_ref[...] = local_ref[...] + lin + sin_part + vtb_ref[...]

    y_blk = y_ref[0]
    nanm = jnp.isnan(y_blk)
    ycl = jnp.where(nanm, jnp.float32(0), y_blk)
    a2 = jnp.concatenate([ycl, nanm.astype(jnp.float32)], axis=1)
    rows = jax.lax.broadcasted_iota(jnp.int32, (2 * _J, 1), 0)
    e_mat = (jnp.where(rows == j, jnp.float32(1), jnp.float32(0))
             * w2_ref[0:1, :]
             + jnp.where(rows == j + _J, jnp.float32(1), jnp.float32(0))
             * w2_ref[1:2, :])
    p = jax.lax.dot(a2, e_mat, preferred_element_type=jnp.float32)
    out_vt_ref[0] = base_ref[...] + p


def _space_kernel(space_hbm, out_hbm, row_v, buf_v, sem):
    c = jax.lax.axis_index("c")
    s = jax.lax.axis_index("s")
    wid = s * 2 + c
    b = wid // _J
    j = wid % _J
    pltpu.sync_copy(space_hbm.at[pl.ds(j, 1)], row_v)
    vs = [row_v[0, pl.ds(i * 16, 16)] for i in range(_D // 16)]

    def rep(k, carry):
        for i in range(_D // 16):
            buf_v[k, pl.ds(i * 16, 16)] = vs[i]
        return carry

    jax.lax.fori_loop(0, _K, rep, 0)
    copies = [
        pltpu.async_copy(
            buf_v, out_hbm.at[b, pl.ds(j * _RPW + i * _K, _K)], sem)
        for i in range(_NDMA)
    ]
    for cp in copies:
        cp.wait()


def kernel(y, x, t2v_w, t2v_b, local_table, vt_W, vt_b, given_table, space_table):
    w1 = vt_W[1:]
    a_mat = t2v_w[:, 0:1] * w1[0::_TED]
    c_lin = t2v_b[:, 0] @ w1[0::_TED]
    ss_t = (jnp.eye(_DX, dtype=jnp.float32)[:, :, None]
            * t2v_w[:, None, 1:]).reshape(_DX, _NS).T
    bs_t = t2v_b[:, 1:].reshape(_NS, 1)
    sin_rows = (jnp.arange(_DX * _TED).reshape(_DX, _TED)[:, 1:]).reshape(-1)
    w1s = w1[sin_rows]
    vtb2 = (vt_b + c_lin + given_table[1]).reshape(1, _D)
    w2 = jnp.concatenate(
        [vt_W[0:1], (given_table[0] - given_table[1]).reshape(1, _D)], axis=0)
    x_t = jnp.transpose(x, (0, 2, 1))

    grid = (_B, _J)
    out_vt = pl.pallas_call(
        _vt_kernel,
        grid=grid,
        in_specs=[
            pl.BlockSpec((1, _DX, _L), lambda b, j: (b, 0, 0)),
            pl.BlockSpec((1, _L, _J), lambda b, j: (b, 0, 0)),
            pl.BlockSpec((_DX, _D), lambda b, j: (0, 0)),
            pl.BlockSpec((_NS, _DX), lambda b, j: (0, 0)),
            pl.BlockSpec((_NS, 1), lambda b, j: (0, 0)),
            pl.BlockSpec((_NS, _D), lambda b, j: (0, 0)),
            pl.BlockSpec((2, _D), lambda b, j: (0, 0)),
            pl.BlockSpec((1, _D), lambda b, j: (0, 0)),
            pl.BlockSpec((_L, _D), lambda b, j: (0, 0)),
        ],
        out_specs=pl.BlockSpec((1, _L, _D), lambda b, j: (b, j, 0)),
        out_shape=jax.ShapeDtypeStruct((_B, _J * _L, _D), jnp.float32),
        scratch_shapes=[pltpu.VMEM((_L, _D), jnp.float32)],
        compiler_params=pltpu.CompilerParams(
            dimension_semantics=("parallel", "arbitrary")),
    )(x_t, y, a_mat, ss_t, bs_t, w1s, w2, vtb2, local_table)

    sc_call = pl.kernel(
        _space_kernel,
        out_type=jax.ShapeDtypeStruct((_B, _J * _L, _D), jnp.float32),
        mesh=plsc.VectorSubcoreMesh(core_axis_name="c", subcore_axis_name="s"),
        scratch_types=[
            pltpu.VMEM((1, _D), jnp.float32),
            pltpu.VMEM((_K, _D), jnp.float32),
            pltpu.SemaphoreType.DMA,
        ],
    )
    out_sp = sc_call(space_table)

    var_idx = jnp.broadcast_to(
        jnp.repeat(jnp.arange(_J, dtype=jnp.int32), _L)[None, :],
        (_B, _J * _L))
    return (out_vt, out_sp, var_idx)


# speedup vs baseline: 1.6787x; 1.2495x over previous
import jax
import jax.numpy as jnp
from jax.experimental import pallas as pl
from jax.experimental.pallas import tpu as pltpu
from jax.experimental.pallas import tpu_sc as plsc

_B, _L, _J, _D = 4, 2048, 8, 256
_DX, _TED = 6, 6
_NS = _DX * (_TED - 1)

_NW = 32
_ROWS = _B * _J * _L
_RPW = _ROWS // _NW
_K = 128
_NDMA = _RPW // _K


_DN0 = (((0,), (0,)), ((), ()))


def _vt_kernel(x_ref, y_ref, a_ref, ss_ref, bs_ref, w1s_ref, w2_ref,
               vtb_ref, local_ref, out_vt_ref):
    xt = jnp.nan_to_num(x_ref[0])
    lin = jax.lax.dot_general(xt, a_ref[...], _DN0,
                              preferred_element_type=jnp.float32)
    xs_t = jax.lax.dot(ss_ref[...], xt,
                       preferred_element_type=jnp.float32) + bs_ref[...]
    te_t = jnp.sin(xs_t)
    sin_part = jax.lax.dot_general(te_t, w1s_ref[...], _DN0,
                                   preferred_element_type=jnp.float32)
    base = local_ref[...] + lin + sin_part + vtb_ref[...]

    y_blk = y_ref[0]
    nanm = jnp.isnan(y_blk)
    ycl = jnp.where(nanm, jnp.float32(0), y_blk)
    a2 = jnp.concatenate([ycl, nanm.astype(jnp.float32)], axis=1)
    rows = jax.lax.broadcasted_iota(jnp.int32, (2 * _J, 1), 0)
    for j in range(_J):
        e_mat = (jnp.where(rows == j, jnp.float32(1), jnp.float32(0))
                 * w2_ref[0:1, :]
                 + jnp.where(rows == j + _J, jnp.float32(1), jnp.float32(0))
                 * w2_ref[1:2, :])
        p = jax.lax.dot(a2, e_mat, preferred_element_type=jnp.float32)
        out_vt_ref[0, j * _L:(j + 1) * _L, :] = base + p


def _space_kernel(space_hbm, out_hbm, row_v, buf_v, sem):
    c = jax.lax.axis_index("c")
    s = jax.lax.axis_index("s")
    wid = s * 2 + c
    b = wid // _J
    j = wid % _J
    pltpu.sync_copy(space_hbm.at[pl.ds(j, 1)], row_v)
    vs = [row_v[0, pl.ds(i * 16, 16)] for i in range(_D // 16)]

    def rep(k, carry):
        for i in range(_D // 16):
            buf_v[k, pl.ds(i * 16, 16)] = vs[i]
        return carry

    jax.lax.fori_loop(0, _K, rep, 0)
    copies = [
        pltpu.async_copy(
            buf_v, out_hbm.at[b, pl.ds(j * _RPW + i * _K, _K)], sem)
        for i in range(_NDMA)
    ]
    for cp in copies:
        cp.wait()


def kernel(y, x, t2v_w, t2v_b, local_table, vt_W, vt_b, given_table, space_table):
    w1 = vt_W[1:]
    a_mat = t2v_w[:, 0:1] * w1[0::_TED]
    c_lin = t2v_b[:, 0] @ w1[0::_TED]
    ss_t = (jnp.eye(_DX, dtype=jnp.float32)[:, :, None]
            * t2v_w[:, None, 1:]).reshape(_DX, _NS).T
    bs_t = t2v_b[:, 1:].reshape(_NS, 1)
    sin_rows = (jnp.arange(_DX * _TED).reshape(_DX, _TED)[:, 1:]).reshape(-1)
    w1s = w1[sin_rows]
    vtb2 = (vt_b + c_lin + given_table[1]).reshape(1, _D)
    w2 = jnp.concatenate(
        [vt_W[0:1], (given_table[0] - given_table[1]).reshape(1, _D)], axis=0)
    x_t = jnp.transpose(x, (0, 2, 1))

    grid = (_B,)
    out_vt = pl.pallas_call(
        _vt_kernel,
        grid=grid,
        in_specs=[
            pl.BlockSpec((1, _DX, _L), lambda b: (b, 0, 0)),
            pl.BlockSpec((1, _L, _J), lambda b: (b, 0, 0)),
            pl.BlockSpec((_DX, _D), lambda b: (0, 0)),
            pl.BlockSpec((_NS, _DX), lambda b: (0, 0)),
            pl.BlockSpec((_NS, 1), lambda b: (0, 0)),
            pl.BlockSpec((_NS, _D), lambda b: (0, 0)),
            pl.BlockSpec((2, _D), lambda b: (0, 0)),
            pl.BlockSpec((1, _D), lambda b: (0, 0)),
            pl.BlockSpec((_L, _D), lambda b: (0, 0)),
        ],
        out_specs=pl.BlockSpec((1, _J * _L, _D), lambda b: (b, 0, 0)),
        out_shape=jax.ShapeDtypeStruct((_B, _J * _L, _D), jnp.float32),
        compiler_params=pltpu.CompilerParams(
            dimension_semantics=("arbitrary",)),
    )(x_t, y, a_mat, ss_t, bs_t, w1s, w2, vtb2, local_table)

    sc_call = pl.kernel(
        _space_kernel,
        out_type=jax.ShapeDtypeStruct((_B, _J * _L, _D), jnp.float32),
        mesh=plsc.VectorSubcoreMesh(core_axis_name="c", subcore_axis_name="s"),
        scratch_types=[
            pltpu.VMEM((1, _D), jnp.float32),
            pltpu.VMEM((_K, _D), jnp.float32),
            pltpu.SemaphoreType.DMA,
        ],
    )
    out_sp = sc_call(space_table)

    var_idx = jnp.broadcast_to(
        jnp.repeat(jnp.arange(_J, dtype=jnp.int32), _L)[None, :],
        (_B, _J * _L))
    return (out_vt, out_sp, var_idx)


# speedup vs baseline: 1.7390x; 1.0359x over previous
import jax
import jax.numpy as jnp
from jax.experimental import pallas as pl
from jax.experimental.pallas import tpu as pltpu
from jax.experimental.pallas import tpu_sc as plsc

_B, _L, _J, _D = 4, 2048, 8, 256
_DX, _TED = 6, 6
_NS = _DX * (_TED - 1)

_NW = 32
_ROWS = _B * _J * _L
_RPW = _ROWS // _NW
_K = 128
_NDMA = _RPW // _K


_DN0 = (((0,), (0,)), ((), ()))


def _vt_kernel(x_ref, y_ref, a_ref, ss_ref, bs_ref, w1s_ref, w2_ref,
               vtb_ref, local_ref, out_vt_ref):
    b = pl.program_id(0)
    xb = x_ref[:, pl.ds(b, 1), :]
    xt = jnp.nan_to_num(jax.lax.squeeze(xb, (1,)))
    lin = jax.lax.dot_general(xt, a_ref[...], _DN0,
                              preferred_element_type=jnp.float32)
    xs_t = jax.lax.dot(ss_ref[...], xt,
                       preferred_element_type=jnp.float32) + bs_ref[...]
    te_t = jnp.sin(xs_t)
    sin_part = jax.lax.dot_general(te_t, w1s_ref[...], _DN0,
                                   preferred_element_type=jnp.float32)
    base = local_ref[...] + lin + sin_part + vtb_ref[...]

    y_jl = y_ref[0]
    nanm = jnp.isnan(y_jl)
    ycl = jnp.where(nanm, jnp.float32(0), y_jl)
    nanf = nanm.astype(jnp.float32)
    for j in range(_J):
        a2 = jnp.concatenate([ycl[j:j + 1], nanf[j:j + 1]], axis=0)
        p = jax.lax.dot_general(a2, w2_ref[...], _DN0,
                                preferred_element_type=jnp.float32)
        out_vt_ref[0, j * _L:(j + 1) * _L, :] = base + p


def _space_kernel(space_hbm, out_hbm, row_v, buf_v, sem):
    c = jax.lax.axis_index("c")
    s = jax.lax.axis_index("s")
    wid = s * 2 + c
    b = wid // _J
    j = wid % _J
    pltpu.sync_copy(space_hbm.at[pl.ds(j, 1)], row_v)
    vs = [row_v[0, pl.ds(i * 16, 16)] for i in range(_D // 16)]

    def rep(k, carry):
        for i in range(_D // 16):
            buf_v[k, pl.ds(i * 16, 16)] = vs[i]
        return carry

    jax.lax.fori_loop(0, _K, rep, 0)
    copies = [
        pltpu.async_copy(
            buf_v, out_hbm.at[b, pl.ds(j * _RPW + i * _K, _K)], sem)
        for i in range(_NDMA)
    ]
    for cp in copies:
        cp.wait()


def kernel(y, x, t2v_w, t2v_b, local_table, vt_W, vt_b, given_table, space_table):
    w1 = vt_W[1:]
    a_mat = t2v_w[:, 0:1] * w1[0::_TED]
    c_lin = t2v_b[:, 0] @ w1[0::_TED]
    ss_t = (jnp.eye(_DX, dtype=jnp.float32)[:, :, None]
            * t2v_w[:, None, 1:]).reshape(_DX, _NS).T
    bs_t = t2v_b[:, 1:].reshape(_NS, 1)
    sin_rows = (jnp.arange(_DX * _TED).reshape(_DX, _TED)[:, 1:]).reshape(-1)
    w1s = w1[sin_rows]
    vtb2 = (vt_b + c_lin + given_table[1]).reshape(1, _D)
    w2 = jnp.concatenate(
        [vt_W[0:1], (given_table[0] - given_table[1]).reshape(1, _D)], axis=0)
    x_t = jnp.transpose(x, (2, 0, 1))
    y_t = jnp.transpose(y, (0, 2, 1))

    grid = (_B,)
    out_vt = pl.pallas_call(
        _vt_kernel,
        grid=grid,
        in_specs=[
            pl.BlockSpec((_DX, _B, _L), lambda b: (0, 0, 0)),
            pl.BlockSpec((1, _J, _L), lambda b: (b, 0, 0)),
            pl.BlockSpec((_DX, _D), lambda b: (0, 0)),
            pl.BlockSpec((_NS, _DX), lambda b: (0, 0)),
            pl.BlockSpec((_NS, 1), lambda b: (0, 0)),
            pl.BlockSpec((_NS, _D), lambda b: (0, 0)),
            pl.BlockSpec((2, _D), lambda b: (0, 0)),
            pl.BlockSpec((1, _D), lambda b: (0, 0)),
            pl.BlockSpec((_L, _D), lambda b: (0, 0)),
        ],
        out_specs=pl.BlockSpec((1, _J * _L, _D), lambda b: (b, 0, 0)),
        out_shape=jax.ShapeDtypeStruct((_B, _J * _L, _D), jnp.float32),
        compiler_params=pltpu.CompilerParams(
            dimension_semantics=("arbitrary",)),
    )(x_t, y_t, a_mat, ss_t, bs_t, w1s, w2, vtb2, local_table)

    sc_call = pl.kernel(
        _space_kernel,
        out_type=jax.ShapeDtypeStruct((_B, _J * _L, _D), jnp.float32),
        mesh=plsc.VectorSubcoreMesh(core_axis_name="c", subcore_axis_name="s"),
        scratch_types=[
            pltpu.VMEM((1, _D), jnp.float32),
            pltpu.VMEM((_K, _D), jnp.float32),
            pltpu.SemaphoreType.DMA,
        ],
    )
    out_sp = sc_call(space_table)

    var_idx = jnp.broadcast_to(
        jnp.repeat(jnp.arange(_J, dtype=jnp.int32), _L)[None, :],
        (_B, _J * _L))
    return (out_vt, out_sp, var_idx)
